# Initial kernel scaffold; baseline (speedup 1.0000x reference)
import jax
import jax.numpy as jnp
from jax.experimental import pallas as pl


def kernel(state, W, b, aval_val, leaf_id):
    raise NotImplementedError("write your pallas kernel here")


# speedup vs baseline: 14.1780x; 14.1780x over previous
import functools

import jax
import jax.numpy as jnp
from jax import lax
from jax.experimental import pallas as pl
from jax.experimental.pallas import tpu as pltpu
from jax.experimental.pallas import tpu_sc as plsc

CHILD = 16
BC_DIM = 3
NODE_TOTAL = 273
BATCH = 4096
STATE_DIM = 2048

_TC_BLOCK = 512


def _tc_logits_body(state_ref, w_ref, b_ref, out_ref):
    x = state_ref[...]
    logits = jnp.dot(x, w_ref[...], preferred_element_type=jnp.float32)
    logits = logits + b_ref[...]
    out_ref[...] = jnp.clip(jax.nn.relu(logits), 1e-30, 1.0)


def _tc_clipped_probs(state, W, b2d):
    grid = state.shape[0] // _TC_BLOCK
    return pl.pallas_call(
        _tc_logits_body,
        grid=(grid,),
        in_specs=[
            pl.BlockSpec((_TC_BLOCK, STATE_DIM), lambda i: (i, 0)),
            pl.BlockSpec((STATE_DIM, CHILD), lambda i: (0, 0)),
            pl.BlockSpec((1, CHILD), lambda i: (0, 0)),
        ],
        out_specs=pl.BlockSpec((_TC_BLOCK, CHILD), lambda i: (i, 0)),
        out_shape=jax.ShapeDtypeStruct((state.shape[0], CHILD), jnp.float32),
    )(state, W, b2d)


def _sc_route(clipped, aval_val, leaf_id):
    info = plsc.get_sparse_core_info()
    nc, ns = info.num_cores, info.num_subcores
    nw = nc * ns
    bpw = BATCH // nw
    mesh = plsc.VectorSubcoreMesh(core_axis_name="c", subcore_axis_name="s")

    @functools.partial(
        pl.kernel,
        mesh=mesh,
        compiler_params=pltpu.CompilerParams(needs_layout_passes=False),
        out_type=(
            jax.ShapeDtypeStruct((BATCH * CHILD,), jnp.float32),
            jax.ShapeDtypeStruct((BATCH,), jnp.int32),
        ),
        scratch_types=[
            pltpu.VMEM((bpw * CHILD,), jnp.float32),
            pltpu.VMEM((CHILD * NODE_TOTAL,), jnp.float32),
            pltpu.VMEM((BATCH,), jnp.int32),
            pltpu.VMEM((bpw * CHILD,), jnp.float32),
            pltpu.VMEM((bpw,), jnp.int32),
        ],
    )
    def route(clip_hbm, aval_hbm, leaf_hbm, mix_hbm, act_hbm,
              clip_v, aval_v, leaf_v, mix_v, act_v):
        wid = lax.axis_index("s") * nc + lax.axis_index("c")
        base = wid * bpw * CHILD
        pltpu.sync_copy(clip_hbm.at[pl.ds(base, bpw * CHILD)], clip_v)
        pltpu.sync_copy(aval_hbm, aval_v)
        pltpu.sync_copy(leaf_hbm, leaf_v)

        lanes = lax.iota(jnp.int32, CHILD)
        lane0 = lanes == 0
        child_stride = lanes * NODE_TOTAL
        a0 = plsc.load_gather(aval_v, [child_stride])
        p0 = a0 / jnp.sum(a0, axis=0)

        def argmax16(v):
            return plsc.all_reduce_ffs(v == jnp.max(v, axis=0))

        def body(i, carry):
            i_v = jnp.full((CHILD,), i, jnp.int32)
            row = plsc.load_gather(clip_v, [i_v * CHILD + lanes])
            c0 = argmax16(row * p0)
            a1 = plsc.load_gather(aval_v, [child_stride + 1 + c0])
            c1 = argmax16(row * (a1 / jnp.sum(a1, axis=0)))
            a2 = plsc.load_gather(aval_v, [child_stride + 17 + CHILD * c0 + c1])
            mix2 = row * (a2 / jnp.sum(a2, axis=0))
            c2 = argmax16(mix2)
            leaf_idx = CHILD * (CHILD * c0 + c1) + c2
            act = plsc.load_gather(leaf_v, [leaf_idx])
            plsc.store_scatter(mix_v, [i_v * CHILD + lanes], mix2)
            plsc.store_scatter(act_v, [i_v], act, mask=lane0)
            return carry

        lax.fori_loop(0, bpw, body, 0)
        pltpu.sync_copy(mix_v, mix_hbm.at[pl.ds(base, bpw * CHILD)])
        pltpu.sync_copy(act_v, act_hbm.at[pl.ds(wid * bpw, bpw)])

    return route(clipped.reshape(-1), aval_val.reshape(-1), leaf_id)


def kernel(state, W, b, aval_val, leaf_id):
    clipped = _tc_clipped_probs(state, W, b.reshape(1, CHILD))
    mix_flat, act = _sc_route(clipped, aval_val, leaf_id)
    return mix_flat.reshape(BATCH, CHILD), act


# speedup vs baseline: 16.4842x; 1.1627x over previous
import functools

import jax
import jax.numpy as jnp
from jax import lax
from jax.experimental import pallas as pl
from jax.experimental.pallas import tpu as pltpu
from jax.experimental.pallas import tpu_sc as plsc

CHILD = 16
NODE_TOTAL = 273
BATCH = 4096
STATE_DIM = 2048

_TC_BLOCK = 512
_KC = 256


def _tc_body(state_ref, w_ref, b_ref, aval_ref, clip_ref, probt_ref):
    acc = jnp.dot(state_ref[:, 0:_KC], w_ref[0:_KC, :],
                  preferred_element_type=jnp.float32)
    for i in range(1, STATE_DIM // _KC):
        acc = acc + jnp.dot(state_ref[:, i * _KC:(i + 1) * _KC],
                            w_ref[i * _KC:(i + 1) * _KC, :],
                            preferred_element_type=jnp.float32)
    logits = acc + b_ref[...]
    clip_ref[...] = jnp.clip(jax.nn.relu(logits), 1e-30, 1.0)

    @pl.when(pl.program_id(0) == 0)
    def _():
        a = aval_ref[...]
        probt_ref[...] = a / jnp.sum(a, axis=0, keepdims=True)


def _tc_stage(state, W, b2d, aval_val):
    grid = state.shape[0] // _TC_BLOCK
    return pl.pallas_call(
        _tc_body,
        grid=(grid,),
        in_specs=[
            pl.BlockSpec((_TC_BLOCK, STATE_DIM), lambda i: (i, 0)),
            pl.BlockSpec((STATE_DIM, CHILD), lambda i: (0, 0)),
            pl.BlockSpec((1, CHILD), lambda i: (0, 0)),
            pl.BlockSpec((CHILD, NODE_TOTAL), lambda i: (0, 0)),
        ],
        out_specs=[
            pl.BlockSpec((_TC_BLOCK, CHILD), lambda i: (i, 0)),
            pl.BlockSpec((CHILD, NODE_TOTAL), lambda i: (0, 0)),
        ],
        out_shape=[
            jax.ShapeDtypeStruct((state.shape[0], CHILD), jnp.float32),
            jax.ShapeDtypeStruct((CHILD, NODE_TOTAL), jnp.float32),
        ],
    )(state, W, b2d, aval_val)


def _sc_route(clipped_flat, probt_flat, leaf_id):
    info = plsc.get_sparse_core_info()
    nc, ns = info.num_cores, info.num_subcores
    nw = nc * ns
    bpw = BATCH // nw
    groups = bpw // CHILD
    mesh = plsc.VectorSubcoreMesh(core_axis_name="c", subcore_axis_name="s")

    @functools.partial(
        pl.kernel,
        mesh=mesh,
        compiler_params=pltpu.CompilerParams(needs_layout_passes=False),
        out_type=(
            jax.ShapeDtypeStruct((BATCH * CHILD,), jnp.float32),
            jax.ShapeDtypeStruct((BATCH,), jnp.int32),
        ),
        scratch_types=[
            pltpu.VMEM((bpw * CHILD,), jnp.float32),
            pltpu.VMEM((CHILD * NODE_TOTAL,), jnp.float32),
            pltpu.VMEM((BATCH,), jnp.int32),
            pltpu.VMEM((bpw * CHILD,), jnp.float32),
            pltpu.VMEM((bpw,), jnp.int32),
            pltpu.SemaphoreType.DMA,
            pltpu.SemaphoreType.DMA,
            pltpu.SemaphoreType.DMA,
        ],
    )
    def route(clip_hbm, probt_hbm, leaf_hbm, mix_hbm, act_hbm,
              clip_v, probt_v, leaf_v, mix_v, act_v, sem0, sem1, sem2):
        wid = lax.axis_index("s") * nc + lax.axis_index("c")
        base = wid * bpw * CHILD
        cp0 = pltpu.async_copy(clip_hbm.at[pl.ds(base, bpw * CHILD)], clip_v, sem0)
        cp1 = pltpu.async_copy(probt_hbm, probt_v, sem1)
        cp2 = pltpu.async_copy(leaf_hbm, leaf_v, sem2)
        cp0.wait()
        cp1.wait()
        cp2.wait()

        lanes = lax.iota(jnp.int32, CHILD)
        p0vec = plsc.load_gather(probt_v, [lanes * NODE_TOTAL])
        p0 = [p0vec[j] for j in range(CHILD)]

        for g in range(groups):
            sidx = (g * CHILD + lanes) * CHILD
            rows = [plsc.load_gather(clip_v, [sidx + j]) for j in range(CHILD)]

            m = rows[0] * p0[0]
            c0 = jnp.zeros((CHILD,), jnp.int32)
            for j in range(1, CHILD):
                v = rows[j] * p0[j]
                gt = v > m
                c0 = jnp.where(gt, jnp.int32(j), c0)
                m = jnp.where(gt, v, m)

            n1 = 1 + c0
            m = rows[0] * plsc.load_gather(probt_v, [n1])
            c1 = jnp.zeros((CHILD,), jnp.int32)
            for j in range(1, CHILD):
                v = rows[j] * plsc.load_gather(probt_v, [j * NODE_TOTAL + n1])
                gt = v > m
                c1 = jnp.where(gt, jnp.int32(j), c1)
                m = jnp.where(gt, v, m)

            n2 = 17 + CHILD * c0 + c1
            mix0 = rows[0] * plsc.load_gather(probt_v, [n2])
            plsc.store_scatter(mix_v, [sidx], mix0)
            m = mix0
            c2 = jnp.zeros((CHILD,), jnp.int32)
            for j in range(1, CHILD):
                v = rows[j] * plsc.load_gather(probt_v, [j * NODE_TOTAL + n2])
                plsc.store_scatter(mix_v, [sidx + j], v)
                gt = v > m
                c2 = jnp.where(gt, jnp.int32(j), c2)
                m = jnp.where(gt, v, m)

            leaf_idx = CHILD * (CHILD * c0 + c1) + c2
            act = plsc.load_gather(leaf_v, [leaf_idx])
            plsc.store_scatter(act_v, [g * CHILD + lanes], act)

        cpo0 = pltpu.async_copy(mix_v, mix_hbm.at[pl.ds(base, bpw * CHILD)], sem0)
        cpo1 = pltpu.async_copy(act_v, act_hbm.at[pl.ds(wid * bpw, bpw)], sem1)
        cpo0.wait()
        cpo1.wait()

    return route(clipped_flat, probt_flat, leaf_id)


def kernel(state, W, b, aval_val, leaf_id):
    clipped, probt = _tc_stage(state, W, b.reshape(1, CHILD), aval_val)
    mix_flat, act = _sc_route(clipped.reshape(-1), probt.reshape(-1), leaf_id)
    return mix_flat.reshape(BATCH, CHILD), act


# speedup vs baseline: 16.7967x; 1.0190x over previous
import functools

import jax
import jax.numpy as jnp
from jax import lax
from jax.experimental import pallas as pl
from jax.experimental.pallas import tpu as pltpu
from jax.experimental.pallas import tpu_sc as plsc

CHILD = 16
NODE_TOTAL = 273
NODE_PAD = 384
BATCH = 4096
STATE_DIM = 2048
LANE = 128

_TC_BLOCK = 512
_KC = 256


def _tc_body(state_ref, w_ref, b_ref, aval_ref, clip_ref, probt_ref):
    acc = jnp.dot(state_ref[:, 0:_KC], w_ref[0:_KC, :],
                  preferred_element_type=jnp.float32)
    for i in range(1, STATE_DIM // _KC):
        acc = acc + jnp.dot(state_ref[:, i * _KC:(i + 1) * _KC],
                            w_ref[i * _KC:(i + 1) * _KC, :],
                            preferred_element_type=jnp.float32)
    logits = acc + b_ref[...]
    clip_ref[:, 0:CHILD] = jnp.clip(jax.nn.relu(logits), 1e-30, 1.0)

    @pl.when(pl.program_id(0) == 0)
    def _():
        a = aval_ref[...]
        probt_ref[:, 0:NODE_TOTAL] = a / jnp.sum(a, axis=0, keepdims=True)


def _tc_stage(state, W, b2d, aval_val):
    grid = state.shape[0] // _TC_BLOCK
    return pl.pallas_call(
        _tc_body,
        grid=(grid,),
        in_specs=[
            pl.BlockSpec((_TC_BLOCK, STATE_DIM), lambda i: (i, 0)),
            pl.BlockSpec((STATE_DIM, CHILD), lambda i: (0, 0)),
            pl.BlockSpec((1, CHILD), lambda i: (0, 0)),
            pl.BlockSpec((CHILD, NODE_TOTAL), lambda i: (0, 0)),
        ],
        out_specs=[
            pl.BlockSpec((_TC_BLOCK, LANE), lambda i: (i, 0)),
            pl.BlockSpec((CHILD, NODE_PAD), lambda i: (0, 0)),
        ],
        out_shape=[
            jax.ShapeDtypeStruct((state.shape[0], LANE), jnp.float32),
            jax.ShapeDtypeStruct((CHILD, NODE_PAD), jnp.float32),
        ],
    )(state, W, b2d, aval_val)


def _sc_route(clipped_flat, probt_flat, leaf_id):
    info = plsc.get_sparse_core_info()
    nc, ns = info.num_cores, info.num_subcores
    nw = nc * ns
    bpw = BATCH // nw
    groups = bpw // CHILD
    mesh = plsc.VectorSubcoreMesh(core_axis_name="c", subcore_axis_name="s")

    @functools.partial(
        pl.kernel,
        mesh=mesh,
        compiler_params=pltpu.CompilerParams(needs_layout_passes=False),
        out_type=(
            jax.ShapeDtypeStruct((BATCH * CHILD,), jnp.float32),
            jax.ShapeDtypeStruct((BATCH,), jnp.int32),
        ),
        scratch_types=[
            pltpu.VMEM((bpw * LANE,), jnp.float32),
            pltpu.VMEM((CHILD * NODE_PAD,), jnp.float32),
            pltpu.VMEM((BATCH,), jnp.int32),
            pltpu.VMEM((bpw * CHILD,), jnp.float32),
            pltpu.VMEM((bpw,), jnp.int32),
            pltpu.SemaphoreType.DMA,
            pltpu.SemaphoreType.DMA,
            pltpu.SemaphoreType.DMA,
        ],
    )
    def route(clip_hbm, probt_hbm, leaf_hbm, mix_hbm, act_hbm,
              clip_v, probt_v, leaf_v, mix_v, act_v, sem0, sem1, sem2):
        wid = lax.axis_index("s") * nc + lax.axis_index("c")
        base = wid * bpw
        cp0 = pltpu.async_copy(clip_hbm.at[pl.ds(base * LANE, bpw * LANE)], clip_v, sem0)
        cp1 = pltpu.async_copy(probt_hbm, probt_v, sem1)
        cp2 = pltpu.async_copy(leaf_hbm, leaf_v, sem2)
        cp0.wait()
        cp1.wait()
        cp2.wait()

        lanes = lax.iota(jnp.int32, CHILD)
        p0vec = plsc.load_gather(probt_v, [lanes * NODE_PAD])
        p0 = [p0vec[j] for j in range(CHILD)]

        for g in range(groups):
            spad = (g * CHILD + lanes) * LANE
            sidx = (g * CHILD + lanes) * CHILD
            rows = [plsc.load_gather(clip_v, [spad + j]) for j in range(CHILD)]

            m = rows[0] * p0[0]
            c0 = jnp.zeros((CHILD,), jnp.int32)
            for j in range(1, CHILD):
                v = rows[j] * p0[j]
                gt = v > m
                c0 = jnp.where(gt, jnp.int32(j), c0)
                m = jnp.where(gt, v, m)

            n1 = 1 + c0
            m = rows[0] * plsc.load_gather(probt_v, [n1])
            c1 = jnp.zeros((CHILD,), jnp.int32)
            for j in range(1, CHILD):
                v = rows[j] * plsc.load_gather(probt_v, [j * NODE_PAD + n1])
                gt = v > m
                c1 = jnp.where(gt, jnp.int32(j), c1)
                m = jnp.where(gt, v, m)

            n2 = 17 + CHILD * c0 + c1
            mix0 = rows[0] * plsc.load_gather(probt_v, [n2])
            plsc.store_scatter(mix_v, [sidx], mix0)
            m = mix0
            c2 = jnp.zeros((CHILD,), jnp.int32)
            for j in range(1, CHILD):
                v = rows[j] * plsc.load_gather(probt_v, [j * NODE_PAD + n2])
                plsc.store_scatter(mix_v, [sidx + j], v)
                gt = v > m
                c2 = jnp.where(gt, jnp.int32(j), c2)
                m = jnp.where(gt, v, m)

            leaf_idx = CHILD * (CHILD * c0 + c1) + c2
            act = plsc.load_gather(leaf_v, [leaf_idx])
            plsc.store_scatter(act_v, [g * CHILD + lanes], act)

        cpo0 = pltpu.async_copy(mix_v, mix_hbm.at[pl.ds(base * CHILD, bpw * CHILD)], sem0)
        cpo1 = pltpu.async_copy(act_v, act_hbm.at[pl.ds(base, bpw)], sem1)
        cpo0.wait()
        cpo1.wait()

    return route(clipped_flat, probt_flat, leaf_id)


def kernel(state, W, b, aval_val, leaf_id):
    clipped, probt = _tc_stage(state, W, b.reshape(1, CHILD), aval_val)
    mix_flat, act = _sc_route(clipped.reshape(-1), probt.reshape(-1), leaf_id)
    return mix_flat.reshape(BATCH, CHILD), act


# speedup vs baseline: 17.1791x; 1.0228x over previous
import functools

import jax
import jax.numpy as jnp
from jax import lax
from jax.experimental import pallas as pl
from jax.experimental.pallas import tpu as pltpu
from jax.experimental.pallas import tpu_sc as plsc

CHILD = 16
NODE_TOTAL = 273
BATCH = 4096
STATE_DIM = 2048
LANE = 128

_TC_BLOCK = 512
_KC = 256


def _tc_body(state_ref, w_ref, b_ref, aval_ref, clip_ref, probt_ref):
    acc = jnp.dot(state_ref[:, 0:_KC], w_ref[0:_KC, :],
                  preferred_element_type=jnp.float32)
    for i in range(1, STATE_DIM // _KC):
        acc = acc + jnp.dot(state_ref[:, i * _KC:(i + 1) * _KC],
                            w_ref[i * _KC:(i + 1) * _KC, :],
                            preferred_element_type=jnp.float32)
    logits = acc + b_ref[...]
    clip_ref[:, 0:CHILD] = jnp.clip(jax.nn.relu(logits), 1e-30, 1.0)

    @pl.when(pl.program_id(0) == 0)
    def _():
        a = aval_ref[...]
        p = a / jnp.sum(a, axis=0, keepdims=True)
        probt_ref[0:CHILD, :] = p[:, 0:LANE]
        probt_ref[CHILD:2 * CHILD, :] = p[:, LANE:2 * LANE]
        probt_ref[2 * CHILD:3 * CHILD, 0:NODE_TOTAL - 2 * LANE] = p[:, 2 * LANE:NODE_TOTAL]


def _tc_stage(state, W, b2d, aval_val):
    grid = state.shape[0] // _TC_BLOCK
    return pl.pallas_call(
        _tc_body,
        grid=(grid,),
        in_specs=[
            pl.BlockSpec((_TC_BLOCK, STATE_DIM), lambda i: (i, 0)),
            pl.BlockSpec((STATE_DIM, CHILD), lambda i: (0, 0)),
            pl.BlockSpec((1, CHILD), lambda i: (0, 0)),
            pl.BlockSpec((CHILD, NODE_TOTAL), lambda i: (0, 0)),
        ],
        out_specs=[
            pl.BlockSpec((_TC_BLOCK, LANE), lambda i: (i, 0)),
            pl.BlockSpec((3 * CHILD, LANE), lambda i: (0, 0)),
        ],
        out_shape=[
            jax.ShapeDtypeStruct((state.shape[0], LANE), jnp.float32),
            jax.ShapeDtypeStruct((3 * CHILD, LANE), jnp.float32),
        ],
    )(state, W, b2d, aval_val)


def _sc_route(clipped_flat, probt_flat, leaf_id):
    info = plsc.get_sparse_core_info()
    nc, ns = info.num_cores, info.num_subcores
    nw = nc * ns
    bpw = BATCH // nw
    groups = bpw // CHILD
    mesh = plsc.VectorSubcoreMesh(core_axis_name="c", subcore_axis_name="s")

    @functools.partial(
        pl.kernel,
        mesh=mesh,
        compiler_params=pltpu.CompilerParams(needs_layout_passes=False),
        out_type=(
            jax.ShapeDtypeStruct((BATCH * CHILD,), jnp.float32),
            jax.ShapeDtypeStruct((BATCH,), jnp.int32),
        ),
        scratch_types=[
            pltpu.VMEM((bpw * LANE,), jnp.float32),
            pltpu.VMEM((3 * CHILD * LANE,), jnp.float32),
            pltpu.VMEM((BATCH,), jnp.int32),
            pltpu.VMEM((bpw * CHILD,), jnp.float32),
            pltpu.VMEM((bpw,), jnp.int32),
            pltpu.SemaphoreType.DMA,
            pltpu.SemaphoreType.DMA,
            pltpu.SemaphoreType.DMA,
        ],
    )
    def route(clip_hbm, probt_hbm, leaf_hbm, mix_hbm, act_hbm,
              clip_v, probt_v, leaf_v, mix_v, act_v, sem0, sem1, sem2):
        wid = lax.axis_index("s") * nc + lax.axis_index("c")
        base = wid * bpw
        cp0 = pltpu.async_copy(clip_hbm.at[pl.ds(base * LANE, bpw * LANE)], clip_v, sem0)
        cp1 = pltpu.async_copy(probt_hbm, probt_v, sem1)
        cp2 = pltpu.async_copy(leaf_hbm, leaf_v, sem2)
        cp0.wait()
        cp1.wait()
        cp2.wait()

        lanes = lax.iota(jnp.int32, CHILD)
        p0vec = plsc.load_gather(probt_v, [lanes * LANE])
        p0 = [p0vec[j] for j in range(CHILD)]

        for g in range(groups):
            spad = (g * CHILD + lanes) * LANE
            sidx = (g * CHILD + lanes) * CHILD
            rows = [plsc.load_gather(clip_v, [spad + j]) for j in range(CHILD)]

            m = rows[0] * p0[0]
            c0 = jnp.zeros((CHILD,), jnp.int32)
            for j in range(1, CHILD):
                v = rows[j] * p0[j]
                gt = v > m
                c0 = jnp.where(gt, jnp.int32(j), c0)
                m = jnp.where(gt, v, m)

            n1 = 1 + c0
            m = rows[0] * plsc.load_gather(probt_v, [n1])
            c1 = jnp.zeros((CHILD,), jnp.int32)
            for j in range(1, CHILD):
                v = rows[j] * plsc.load_gather(probt_v, [j * LANE + n1])
                gt = v > m
                c1 = jnp.where(gt, jnp.int32(j), c1)
                m = jnp.where(gt, v, m)

            n2 = 17 + CHILD * c0 + c1
            base2 = ((n2 >> 7) << 11) + (n2 & 127)
            mix0 = rows[0] * plsc.load_gather(probt_v, [base2])
            plsc.store_scatter(mix_v, [sidx], mix0)
            m = mix0
            c2 = jnp.zeros((CHILD,), jnp.int32)
            for j in range(1, CHILD):
                v = rows[j] * plsc.load_gather(probt_v, [j * LANE + base2])
                plsc.store_scatter(mix_v, [sidx + j], v)
                gt = v > m
                c2 = jnp.where(gt, jnp.int32(j), c2)
                m = jnp.where(gt, v, m)

            leaf_idx = CHILD * (CHILD * c0 + c1) + c2
            act = plsc.load_gather(leaf_v, [leaf_idx])
            plsc.store_scatter(act_v, [g * CHILD + lanes], act)

        cpo0 = pltpu.async_copy(mix_v, mix_hbm.at[pl.ds(base * CHILD, bpw * CHILD)], sem0)
        cpo1 = pltpu.async_copy(act_v, act_hbm.at[pl.ds(base, bpw)], sem1)
        cpo0.wait()
        cpo1.wait()

    return route(clipped_flat, probt_flat, leaf_id)


def kernel(state, W, b, aval_val, leaf_id):
    clipped, probt = _tc_stage(state, W, b.reshape(1, CHILD), aval_val)
    mix_flat, act = _sc_route(clipped.reshape(-1), probt.reshape(-1), leaf_id)
    return mix_flat.reshape(BATCH, CHILD), act
